# baseline (speedup 1.0000x reference)
import functools

import jax
import jax.numpy as jnp
from jax import lax
from jax.experimental import pallas as pl
from jax.experimental.pallas import tpu as pltpu
from jax.experimental.pallas import tpu_sc as plsc

N_WORKERS = 32
CHUNK = 80
BE = 2560


def _sc_gather(atom_rows, idx0, idx1):
    E = idx0.shape[0]
    W = atom_rows.shape[1]
    dt = atom_rows.dtype
    per_w = E // N_WORKERS
    n_chunks = per_w // CHUNK
    mesh = plsc.VectorSubcoreMesh(core_axis_name="c", subcore_axis_name="s")

    def body(atom_hbm, idx0_hbm, idx1_hbm, out0_hbm, out1_hbm,
             idx0_v, idx1_v, rows0_v, rows1_v, sem0, sem1):
        cid = lax.axis_index("c")
        sid = lax.axis_index("s")
        wid = sid * 2 + cid
        base_w = wid * per_w

        def step(j, carry):
            base = base_w + j * CHUNK
            pltpu.sync_copy(idx0_hbm.at[pl.ds(base, CHUNK)], idx0_v)
            pltpu.sync_copy(idx1_hbm.at[pl.ds(base, CHUNK)], idx1_v)
            c0 = pltpu.async_copy(atom_hbm.at[idx0_v], rows0_v, sem0)
            c1 = pltpu.async_copy(atom_hbm.at[idx1_v], rows1_v, sem1)
            c0.wait()
            c1.wait()
            pltpu.sync_copy(rows0_v, out0_hbm.at[pl.ds(base, CHUNK)])
            pltpu.sync_copy(rows1_v, out1_hbm.at[pl.ds(base, CHUNK)])
            return carry

        lax.fori_loop(0, n_chunks, step, 0)

    k = pl.kernel(
        body,
        out_type=(jax.ShapeDtypeStruct((E, W), dt),
                  jax.ShapeDtypeStruct((E, W), dt)),
        mesh=mesh,
        compiler_params=pltpu.CompilerParams(use_tc_tiling_on_sc=False),
        scratch_types=[
            pltpu.VMEM((CHUNK,), jnp.int32),
            pltpu.VMEM((CHUNK,), jnp.int32),
            pltpu.VMEM((CHUNK, W), dt),
            pltpu.VMEM((CHUNK, W), dt),
            pltpu.SemaphoreType.DMA,
            pltpu.SemaphoreType.DMA,
        ],
    )
    return k(atom_rows, idx0, idx1)


def _sigmoid(x):
    return 0.5 * jnp.tanh(0.5 * x) + 0.5


def _silu(x):
    return x * _sigmoid(x)


def _tc_mlp_body(s_ref, r_ref, bd_ref, w1a, w1b, w1c, b1r, w2, b2r,
                 g1a, g1b, g1c, gb1r, g2w, gb2r, o_ref):
    s = s_ref[...]
    r = r_ref[...]
    bd = bd_ref[...]
    pre_h = (jnp.dot(s, w1a[...], preferred_element_type=jnp.float32)
             + jnp.dot(r, w1b[...], preferred_element_type=jnp.float32)
             + jnp.dot(bd, w1c[...], preferred_element_type=jnp.float32)
             + b1r[...])
    h = _silu(pre_h).astype(jnp.bfloat16)
    h2 = _silu(jnp.dot(h, w2[...], preferred_element_type=jnp.float32) + b2r[...])
    pre_g = (jnp.dot(s, g1a[...], preferred_element_type=jnp.float32)
             + jnp.dot(r, g1b[...], preferred_element_type=jnp.float32)
             + jnp.dot(bd, g1c[...], preferred_element_type=jnp.float32)
             + gb1r[...])
    g = _silu(pre_g).astype(jnp.bfloat16)
    g2 = _sigmoid(
        jnp.dot(g, g2w[...], preferred_element_type=jnp.float32) + gb2r[...])
    o_ref[...] = h2 * g2


def _tc_mlp(sender, receiver, bond, W1a, W1b, W1c, b1, W2, b2,
            G1a, G1b, G1c, gb1, G2, gb2):
    E, D = sender.shape
    DE = bond.shape[1]
    DH = W1a.shape[1]
    DO = W2.shape[1]
    grid = (E // BE,)

    def blk(shape):
        return pl.BlockSpec(shape, lambda i: (i, 0))

    def full(shape):
        return pl.BlockSpec(shape, lambda i: (0, 0))

    return pl.pallas_call(
        _tc_mlp_body,
        grid=grid,
        in_specs=[
            blk((BE, D)), blk((BE, D)), blk((BE, DE)),
            full((D, DH)), full((D, DH)), full((DE, DH)), full((1, DH)),
            full((DH, DO)), full((1, DO)),
            full((D, DH)), full((D, DH)), full((DE, DH)), full((1, DH)),
            full((DH, DO)), full((1, DO)),
        ],
        out_specs=blk((BE, DO)),
        out_shape=jax.ShapeDtypeStruct((E, DO), jnp.float32),
    )(sender, receiver, bond, W1a, W1b, W1c, b1, W2, b2,
      G1a, G1b, G1c, gb1, G2, gb2)


def kernel(atom_features, bond_features, bond_atom_indices,
           W1, b1, W2, b2, G1, gb1, G2, gb2):
    D = atom_features.shape[1]
    idx0 = bond_atom_indices[:, 0]
    idx1 = bond_atom_indices[:, 1]
    atom_bf = atom_features.astype(jnp.bfloat16)
    atom_packed = lax.bitcast_convert_type(
        atom_bf.reshape(atom_bf.shape[0], D // 2, 2), jnp.int32)
    s_packed, r_packed = _sc_gather(atom_packed, idx0, idx1)
    sender = lax.bitcast_convert_type(s_packed, jnp.bfloat16).reshape(-1, D)
    receiver = lax.bitcast_convert_type(r_packed, jnp.bfloat16).reshape(-1, D)
    bf = jnp.bfloat16
    W1a, W1b, W1c = W1[:D].astype(bf), W1[D:2 * D].astype(bf), W1[2 * D:].astype(bf)
    G1a, G1b, G1c = G1[:D].astype(bf), G1[D:2 * D].astype(bf), G1[2 * D:].astype(bf)
    return _tc_mlp(sender, receiver, bond_features.astype(bf),
                   W1a, W1b, W1c, b1[None, :], W2.astype(bf), b2[None, :],
                   G1a, G1b, G1c, gb1[None, :], G2.astype(bf), gb2[None, :])


# speedup vs baseline: 2.4129x; 2.4129x over previous
import functools

import jax
import jax.numpy as jnp
from jax import lax
from jax.experimental import pallas as pl
from jax.experimental.pallas import tpu as pltpu
from jax.experimental.pallas import tpu_sc as plsc

N_WORKERS = 32
CHUNK = 80
BE = 2560


def _sc_gather(atom_rows, idx0, idx1):
    E = idx0.shape[0]
    W = atom_rows.shape[1]
    dt = atom_rows.dtype
    per_w = E // N_WORKERS
    n_chunks = per_w // CHUNK
    mesh = plsc.VectorSubcoreMesh(core_axis_name="c", subcore_axis_name="s")

    def body(atom_hbm, idx0_hbm, idx1_hbm, out0_hbm, out1_hbm,
             idx0_v, idx1_v, rows0_v, rows1_v, sem0, sem1):
        cid = lax.axis_index("c")
        sid = lax.axis_index("s")
        wid = sid * 2 + cid
        base_w = wid * per_w

        def step(j, carry):
            base = base_w + j * CHUNK
            pltpu.sync_copy(idx0_hbm.at[pl.ds(base, CHUNK)], idx0_v)
            pltpu.sync_copy(idx1_hbm.at[pl.ds(base, CHUNK)], idx1_v)
            c0 = pltpu.async_copy(atom_hbm.at[idx0_v], rows0_v, sem0)
            c1 = pltpu.async_copy(atom_hbm.at[idx1_v], rows1_v, sem1)
            c0.wait()
            c1.wait()
            pltpu.sync_copy(rows0_v, out0_hbm.at[pl.ds(base, CHUNK)])
            pltpu.sync_copy(rows1_v, out1_hbm.at[pl.ds(base, CHUNK)])
            return carry

        lax.fori_loop(0, n_chunks, step, 0)

    k = pl.kernel(
        body,
        out_type=(jax.ShapeDtypeStruct((E, W), dt),
                  jax.ShapeDtypeStruct((E, W), dt)),
        mesh=mesh,
        scratch_types=[
            pltpu.VMEM((CHUNK,), jnp.int32),
            pltpu.VMEM((CHUNK,), jnp.int32),
            pltpu.VMEM((CHUNK, W), dt),
            pltpu.VMEM((CHUNK, W), dt),
            pltpu.SemaphoreType.DMA,
            pltpu.SemaphoreType.DMA,
        ],
    )
    return k(atom_rows, idx0, idx1)


def _sigmoid(x):
    return 0.5 * jnp.tanh(0.5 * x) + 0.5


def _silu(x):
    return x * _sigmoid(x)


def _tc_mlp_body(s_ref, r_ref, bd_ref, w1a, w1b, w1c, b1r, w2, b2r,
                 g1a, g1b, g1c, gb1r, g2w, gb2r, o_ref):
    s = s_ref[...].astype(jnp.bfloat16)
    r = r_ref[...].astype(jnp.bfloat16)
    bd = bd_ref[...].astype(jnp.bfloat16)
    pre_h = (jnp.dot(s, w1a[...], preferred_element_type=jnp.float32)
             + jnp.dot(r, w1b[...], preferred_element_type=jnp.float32)
             + jnp.dot(bd, w1c[...], preferred_element_type=jnp.float32)
             + b1r[...])
    h = _silu(pre_h).astype(jnp.bfloat16)
    h2 = _silu(jnp.dot(h, w2[...], preferred_element_type=jnp.float32) + b2r[...])
    pre_g = (jnp.dot(s, g1a[...], preferred_element_type=jnp.float32)
             + jnp.dot(r, g1b[...], preferred_element_type=jnp.float32)
             + jnp.dot(bd, g1c[...], preferred_element_type=jnp.float32)
             + gb1r[...])
    g = _silu(pre_g).astype(jnp.bfloat16)
    g2 = _sigmoid(
        jnp.dot(g, g2w[...], preferred_element_type=jnp.float32) + gb2r[...])
    o_ref[...] = h2 * g2


def _tc_mlp(sender, receiver, bond, W1a, W1b, W1c, b1, W2, b2,
            G1a, G1b, G1c, gb1, G2, gb2):
    E, D = sender.shape
    DE = bond.shape[1]
    DH = W1a.shape[1]
    DO = W2.shape[1]
    grid = (E // BE,)

    def blk(shape):
        return pl.BlockSpec(shape, lambda i: (i, 0))

    def full(shape):
        return pl.BlockSpec(shape, lambda i: (0, 0))

    return pl.pallas_call(
        _tc_mlp_body,
        grid=grid,
        in_specs=[
            blk((BE, D)), blk((BE, D)), blk((BE, DE)),
            full((D, DH)), full((D, DH)), full((DE, DH)), full((1, DH)),
            full((DH, DO)), full((1, DO)),
            full((D, DH)), full((D, DH)), full((DE, DH)), full((1, DH)),
            full((DH, DO)), full((1, DO)),
        ],
        out_specs=blk((BE, DO)),
        out_shape=jax.ShapeDtypeStruct((E, DO), jnp.float32),
    )(sender, receiver, bond, W1a, W1b, W1c, b1, W2, b2,
      G1a, G1b, G1c, gb1, G2, gb2)


def kernel(atom_features, bond_features, bond_atom_indices,
           W1, b1, W2, b2, G1, gb1, G2, gb2):
    D = atom_features.shape[1]
    idx0 = bond_atom_indices[:, 0]
    idx1 = bond_atom_indices[:, 1]
    sender, receiver = _sc_gather(atom_features, idx0, idx1)
    bf = jnp.bfloat16
    W1a, W1b, W1c = W1[:D].astype(bf), W1[D:2 * D].astype(bf), W1[2 * D:].astype(bf)
    G1a, G1b, G1c = G1[:D].astype(bf), G1[D:2 * D].astype(bf), G1[2 * D:].astype(bf)
    return _tc_mlp(sender, receiver, bond_features,
                   W1a, W1b, W1c, b1[None, :], W2.astype(bf), b2[None, :],
                   G1a, G1b, G1c, gb1[None, :], G2.astype(bf), gb2[None, :])
